# baseline (device time: 26701 ns/iter reference)
import ml_dtypes
import numpy as np

import jax
import jax.numpy as jnp
from jax import lax
from jax.experimental import pallas as pl
from jax.experimental.pallas import tpu as pltpu

NZ = 4
T = 1024
D = 512
P_PAD = 288

_TRI = np.triu(np.ones((T, T), dtype=ml_dtypes.bfloat16), k=1)


def kernel(x, dest):
    d_row = dest.reshape(1, T)
    d_col = dest.reshape(T, 1)
    tri = jnp.asarray(_TRI)

    def body(x_ref, drow_ref, dcol_ref, tri_ref, out_ref,
             send_ref, recv_ref, csend_ref, crecv_ref,
             csend_sems, crecv_sems, bsend_sems, brecv_sems):
        my_x = lax.axis_index("x")
        my_y = lax.axis_index("y")
        my_z = lax.axis_index("z")

        barrier_sem = pltpu.get_barrier_semaphore()
        for off in range(1, NZ):
            zp = lax.rem(my_z + off, NZ)
            pl.semaphore_signal(
                barrier_sem, inc=1,
                device_id=(my_x, my_y, zp),
                device_id_type=pl.DeviceIdType.MESH,
            )

        dcol = dcol_ref[:, :]
        lane = lax.broadcasted_iota(jnp.int32, (T, 128), 1)
        cnt_row = jnp.sum((dcol == lane).astype(jnp.int32), axis=0,
                          keepdims=True)
        csend_ref[:, :] = cnt_row
        for d in range(NZ):
            @pl.when(d == my_z)
            def _():
                crecv_ref[d, :, :] = cnt_row

        drow = drow_ref[:, :]
        x_bf = x_ref[:, :].astype(jnp.bfloat16)

        m8 = (lax.broadcasted_iota(jnp.int32, (8, T), 0) == drow)
        ranks8 = jnp.dot(m8.astype(jnp.bfloat16), tri_ref[:, :],
                         preferred_element_type=jnp.float32)
        rank = jnp.zeros((1, T), jnp.float32)
        for d in range(NZ):
            rank = rank + jnp.where(drow == d, ranks8[d:d + 1, :], 0.0)
        key = drow * P_PAD + rank.astype(jnp.int32)

        srow = lax.broadcasted_iota(jnp.int32, (NZ * P_PAD, T), 0)
        s_all = (srow == key).astype(jnp.bfloat16)
        packed = jnp.dot(s_all, x_bf,
                         preferred_element_type=jnp.float32)
        send_ref[:, :] = packed.astype(jnp.bfloat16)

        for d in range(NZ):
            @pl.when(d == my_z)
            def _():
                recv_ref[d * P_PAD:(d + 1) * P_PAD, :] = \
                    send_ref[d * P_PAD:(d + 1) * P_PAD, :]

        pl.semaphore_wait(barrier_sem, NZ - 1)

        cnt_sends = []
        for k in range(1, NZ):
            d = my_z ^ k
            rdma = pltpu.make_async_remote_copy(
                src_ref=csend_ref,
                dst_ref=crecv_ref.at[my_z],
                send_sem=csend_sems.at[d],
                recv_sem=crecv_sems.at[my_z],
                device_id=(my_x, my_y, d),
                device_id_type=pl.DeviceIdType.MESH,
            )
            rdma.start()
            cnt_sends.append(rdma)

        blk_sends = []
        for k in (3, 2, 1):
            d = my_z ^ k
            rdma = pltpu.make_async_remote_copy(
                src_ref=send_ref.at[pl.ds(d * P_PAD, P_PAD), :],
                dst_ref=recv_ref.at[pl.ds(my_z * P_PAD, P_PAD), :],
                send_sem=bsend_sems.at[d],
                recv_sem=brecv_sems.at[my_z],
                device_id=(my_x, my_y, d),
                device_id_type=pl.DeviceIdType.MESH,
            )
            rdma.start()
            blk_sends.append(rdma)

        for k in range(1, NZ):
            s = my_z ^ k
            pltpu.make_async_remote_copy(
                src_ref=csend_ref,
                dst_ref=crecv_ref.at[s],
                send_sem=csend_sems.at[s],
                recv_sem=crecv_sems.at[s],
                device_id=(my_x, my_y, s),
                device_id_type=pl.DeviceIdType.MESH,
            ).wait_recv()

        lane1 = lax.broadcasted_iota(jnp.int32, (1, 128), 1)
        sel_me = (lane1 == my_z)
        cnts = []
        for s in range(NZ):
            row = crecv_ref[s, :, :]
            cnts.append(jnp.sum(jnp.where(sel_me, row, 0)))
        offs = [jnp.int32(0)]
        for s in range(NZ - 1):
            offs.append(offs[-1] + cnts[s])

        jcol = lax.broadcasted_iota(jnp.int32, (T, 1), 0)
        qkey = jnp.zeros((T, 1), jnp.int32)
        for s in range(NZ):
            in_s = (jcol >= offs[s]) & (jcol < offs[s] + cnts[s])
            qkey = qkey + jnp.where(in_s, s * P_PAD + jcol - offs[s], 0)

        klane = lax.broadcasted_iota(jnp.int32, (T, NZ * P_PAD), 1)
        q_all = (klane == qkey).astype(jnp.bfloat16)

        for k in range(1, NZ):
            s = my_z ^ k
            pltpu.make_async_remote_copy(
                src_ref=send_ref.at[pl.ds(s * P_PAD, P_PAD), :],
                dst_ref=recv_ref.at[pl.ds(s * P_PAD, P_PAD), :],
                send_sem=bsend_sems.at[s],
                recv_sem=brecv_sems.at[s],
                device_id=(my_x, my_y, s),
                device_id_type=pl.DeviceIdType.MESH,
            ).wait_recv()

        out_ref[:, :] = jnp.dot(q_all, recv_ref[:, :],
                                preferred_element_type=jnp.float32)

        for rdma in cnt_sends + blk_sends:
            rdma.wait_send()

    return pl.pallas_call(
        body,
        out_shape=jax.ShapeDtypeStruct((T, D), jnp.float32),
        in_specs=[
            pl.BlockSpec(memory_space=pltpu.VMEM),
            pl.BlockSpec(memory_space=pltpu.VMEM),
            pl.BlockSpec(memory_space=pltpu.VMEM),
            pl.BlockSpec(memory_space=pltpu.VMEM),
        ],
        out_specs=pl.BlockSpec(memory_space=pltpu.VMEM),
        scratch_shapes=[
            pltpu.VMEM((NZ * P_PAD, D), jnp.bfloat16),
            pltpu.VMEM((NZ * P_PAD, D), jnp.bfloat16),
            pltpu.VMEM((1, 128), jnp.int32),
            pltpu.VMEM((NZ, 1, 128), jnp.int32),
            pltpu.SemaphoreType.DMA((NZ,)),
            pltpu.SemaphoreType.DMA((NZ,)),
            pltpu.SemaphoreType.DMA((NZ,)),
            pltpu.SemaphoreType.DMA((NZ,)),
        ],
        compiler_params=pltpu.CompilerParams(collective_id=0),
    )(x, d_row, d_col, tri)


# device time: 26297 ns/iter; 1.0154x vs baseline; 1.0154x over previous
import ml_dtypes
import numpy as np

import jax
import jax.numpy as jnp
from jax import lax
from jax.experimental import pallas as pl
from jax.experimental.pallas import tpu as pltpu

NZ = 4
T = 1024
D = 512
P_PAD = 288

_TRI = np.triu(np.ones((T, T), dtype=ml_dtypes.bfloat16), k=1)


def kernel(x, dest):
    d_row = dest.reshape(1, T)
    d_col = dest.reshape(T, 1)
    tri = jnp.asarray(_TRI)

    def body(x_ref, drow_ref, dcol_ref, tri_ref, out_ref,
             send_ref, recv_ref, csend_ref, crecv_ref,
             csend_sems, crecv_sems, bsend_sems, brecv_sems):
        my_x = lax.axis_index("x")
        my_y = lax.axis_index("y")
        my_z = lax.axis_index("z")

        barrier_sem = pltpu.get_barrier_semaphore()
        for off in range(1, NZ):
            zp = lax.rem(my_z + off, NZ)
            pl.semaphore_signal(
                barrier_sem, inc=1,
                device_id=(my_x, my_y, zp),
                device_id_type=pl.DeviceIdType.MESH,
            )

        dcol = dcol_ref[:, :]
        lane = lax.broadcasted_iota(jnp.int32, (T, 128), 1)
        cnt_row = jnp.sum((dcol == lane).astype(jnp.int32), axis=0,
                          keepdims=True)
        csend_ref[:, :] = cnt_row
        for d in range(NZ):
            @pl.when(d == my_z)
            def _():
                crecv_ref[d, :, :] = cnt_row

        drow = drow_ref[:, :]
        x_bf = x_ref[:, :].astype(jnp.bfloat16)

        m8 = (lax.broadcasted_iota(jnp.int32, (8, T), 0) == drow)
        ranks8 = jnp.dot(m8.astype(jnp.bfloat16), tri_ref[:, :],
                         preferred_element_type=jnp.float32)
        rank = jnp.zeros((1, T), jnp.float32)
        for d in range(NZ):
            rank = rank + jnp.where(drow == d, ranks8[d:d + 1, :], 0.0)
        key = drow * P_PAD + rank.astype(jnp.int32)
        srow = lax.broadcasted_iota(jnp.int32, (P_PAD, T), 0)

        pl.semaphore_wait(barrier_sem, NZ - 1)

        cnt_sends = []
        for k in range(1, NZ):
            d = my_z ^ k
            rdma = pltpu.make_async_remote_copy(
                src_ref=csend_ref,
                dst_ref=crecv_ref.at[my_z],
                send_sem=csend_sems.at[d],
                recv_sem=crecv_sems.at[my_z],
                device_id=(my_x, my_y, d),
                device_id_type=pl.DeviceIdType.MESH,
            )
            rdma.start()
            cnt_sends.append(rdma)

        blk_sends = []
        for k in (3, 2, 1):
            d = my_z ^ k
            s_d = (srow == key - d * P_PAD).astype(jnp.bfloat16)
            packed = jnp.dot(s_d, x_bf,
                             preferred_element_type=jnp.float32)
            send_ref[pl.ds(d * P_PAD, P_PAD), :] = \
                packed.astype(jnp.bfloat16)
            rdma = pltpu.make_async_remote_copy(
                src_ref=send_ref.at[pl.ds(d * P_PAD, P_PAD), :],
                dst_ref=recv_ref.at[pl.ds(my_z * P_PAD, P_PAD), :],
                send_sem=bsend_sems.at[d],
                recv_sem=brecv_sems.at[my_z],
                device_id=(my_x, my_y, d),
                device_id_type=pl.DeviceIdType.MESH,
            )
            rdma.start()
            blk_sends.append(rdma)

        s_me = (srow == key - my_z * P_PAD).astype(jnp.bfloat16)
        packed_me = jnp.dot(s_me, x_bf,
                            preferred_element_type=jnp.float32)
        recv_ref[pl.ds(my_z * P_PAD, P_PAD), :] = \
            packed_me.astype(jnp.bfloat16)

        for k in range(1, NZ):
            s = my_z ^ k
            pltpu.make_async_remote_copy(
                src_ref=csend_ref,
                dst_ref=crecv_ref.at[s],
                send_sem=csend_sems.at[s],
                recv_sem=crecv_sems.at[s],
                device_id=(my_x, my_y, s),
                device_id_type=pl.DeviceIdType.MESH,
            ).wait_recv()

        lane1 = lax.broadcasted_iota(jnp.int32, (1, 128), 1)
        sel_me = (lane1 == my_z)
        cnts = []
        for s in range(NZ):
            row = crecv_ref[s, :, :]
            cnts.append(jnp.sum(jnp.where(sel_me, row, 0)))
        offs = [jnp.int32(0)]
        for s in range(NZ - 1):
            offs.append(offs[-1] + cnts[s])

        jcol = lax.broadcasted_iota(jnp.int32, (T, 1), 0)
        qkey = jnp.zeros((T, 1), jnp.int32)
        for s in range(NZ):
            in_s = (jcol >= offs[s]) & (jcol < offs[s] + cnts[s])
            qkey = qkey + jnp.where(in_s, s * P_PAD + jcol - offs[s], 0)

        klane = lax.broadcasted_iota(jnp.int32, (T, P_PAD), 1)
        acc = jnp.zeros((T, D), jnp.float32)
        for k in range(NZ):
            s = my_z ^ k
            if k > 0:
                pltpu.make_async_remote_copy(
                    src_ref=send_ref.at[pl.ds(s * P_PAD, P_PAD), :],
                    dst_ref=recv_ref.at[pl.ds(s * P_PAD, P_PAD), :],
                    send_sem=bsend_sems.at[s],
                    recv_sem=brecv_sems.at[s],
                    device_id=(my_x, my_y, s),
                    device_id_type=pl.DeviceIdType.MESH,
                ).wait_recv()
            q_s = (klane == qkey - s * P_PAD).astype(jnp.bfloat16)
            blk = recv_ref[pl.ds(s * P_PAD, P_PAD), :]
            acc = acc + jnp.dot(q_s, blk,
                                preferred_element_type=jnp.float32)
        out_ref[:, :] = acc

        for rdma in cnt_sends + blk_sends:
            rdma.wait_send()

    return pl.pallas_call(
        body,
        out_shape=jax.ShapeDtypeStruct((T, D), jnp.float32),
        in_specs=[
            pl.BlockSpec(memory_space=pltpu.VMEM),
            pl.BlockSpec(memory_space=pltpu.VMEM),
            pl.BlockSpec(memory_space=pltpu.VMEM),
            pl.BlockSpec(memory_space=pltpu.VMEM),
        ],
        out_specs=pl.BlockSpec(memory_space=pltpu.VMEM),
        scratch_shapes=[
            pltpu.VMEM((NZ * P_PAD, D), jnp.bfloat16),
            pltpu.VMEM((NZ * P_PAD, D), jnp.bfloat16),
            pltpu.VMEM((1, 128), jnp.int32),
            pltpu.VMEM((NZ, 1, 128), jnp.int32),
            pltpu.SemaphoreType.DMA((NZ,)),
            pltpu.SemaphoreType.DMA((NZ,)),
            pltpu.SemaphoreType.DMA((NZ,)),
            pltpu.SemaphoreType.DMA((NZ,)),
        ],
        compiler_params=pltpu.CompilerParams(collective_id=0),
    )(x, d_row, d_col, tri)


# device time: 21871 ns/iter; 1.2208x vs baseline; 1.2024x over previous
import jax
import jax.numpy as jnp
from jax import lax
from jax.experimental import pallas as pl
from jax.experimental.pallas import tpu as pltpu

NZ = 4
T = 1024
D = 512
P_PAD = 288


def kernel(x, dest):
    d_row = dest.reshape(1, T)
    d_col = dest.reshape(T, 1)

    def body(x_ref, drow_ref, dcol_ref, out_ref,
             send_ref, recv_ref, csend_ref, crecv_ref,
             csend_sems, crecv_sems, bsend_sems, brecv_sems):
        my_x = lax.axis_index("x")
        my_y = lax.axis_index("y")
        my_z = lax.axis_index("z")

        barrier_sem = pltpu.get_barrier_semaphore()
        for off in range(1, NZ):
            zp = lax.rem(my_z + off, NZ)
            pl.semaphore_signal(
                barrier_sem, inc=1,
                device_id=(my_x, my_y, zp),
                device_id_type=pl.DeviceIdType.MESH,
            )

        dcol = dcol_ref[:, :]
        lane = lax.broadcasted_iota(jnp.int32, (T, 128), 1)
        cnt_row = jnp.sum((dcol == lane).astype(jnp.int32), axis=0,
                          keepdims=True)
        csend_ref[:, :] = cnt_row
        for d in range(NZ):
            @pl.when(d == my_z)
            def _():
                crecv_ref[d, :, :] = cnt_row

        drow = drow_ref[:, :]
        x_bf = x_ref[:, :].astype(jnp.bfloat16)

        ii = lax.broadcasted_iota(jnp.int32, (T, T), 0)
        jj = lax.broadcasted_iota(jnp.int32, (T, T), 1)
        tri = (ii < jj).astype(jnp.bfloat16)
        m8 = (lax.broadcasted_iota(jnp.int32, (8, T), 0) == drow)
        ranks8 = jnp.dot(m8.astype(jnp.bfloat16), tri,
                         preferred_element_type=jnp.float32)
        rank = jnp.zeros((1, T), jnp.float32)
        for d in range(NZ):
            rank = rank + jnp.where(drow == d, ranks8[d:d + 1, :], 0.0)
        key = drow * P_PAD + rank.astype(jnp.int32)
        srow = lax.broadcasted_iota(jnp.int32, (P_PAD, T), 0)

        pl.semaphore_wait(barrier_sem, NZ - 1)

        cnt_sends = []
        for off in range(1, NZ):
            d = lax.rem(my_z + off, NZ)
            rdma = pltpu.make_async_remote_copy(
                src_ref=csend_ref,
                dst_ref=crecv_ref.at[my_z],
                send_sem=csend_sems.at[d],
                recv_sem=crecv_sems.at[my_z],
                device_id=(my_x, my_y, d),
                device_id_type=pl.DeviceIdType.MESH,
            )
            rdma.start()
            cnt_sends.append(rdma)

        blk_sends = []
        for off in range(1, NZ):
            d = lax.rem(my_z + off, NZ)
            s_d = (srow == key - d * P_PAD).astype(jnp.bfloat16)
            packed = jnp.dot(s_d, x_bf,
                             preferred_element_type=jnp.float32)
            send_ref[pl.ds(d * P_PAD, P_PAD), :] = \
                packed.astype(jnp.bfloat16)
            rdma = pltpu.make_async_remote_copy(
                src_ref=send_ref.at[pl.ds(d * P_PAD, P_PAD), :],
                dst_ref=recv_ref.at[pl.ds(my_z * P_PAD, P_PAD), :],
                send_sem=bsend_sems.at[d],
                recv_sem=brecv_sems.at[my_z],
                device_id=(my_x, my_y, d),
                device_id_type=pl.DeviceIdType.MESH,
            )
            rdma.start()
            blk_sends.append(rdma)

        s_me = (srow == key - my_z * P_PAD).astype(jnp.bfloat16)
        packed_me = jnp.dot(s_me, x_bf,
                            preferred_element_type=jnp.float32)
        recv_ref[pl.ds(my_z * P_PAD, P_PAD), :] = \
            packed_me.astype(jnp.bfloat16)

        for off in range(1, NZ):
            s = lax.rem(my_z + off, NZ)
            pltpu.make_async_remote_copy(
                src_ref=csend_ref,
                dst_ref=crecv_ref.at[s],
                send_sem=csend_sems.at[s],
                recv_sem=crecv_sems.at[s],
                device_id=(my_x, my_y, s),
                device_id_type=pl.DeviceIdType.MESH,
            ).wait_recv()

        lane1 = lax.broadcasted_iota(jnp.int32, (1, 128), 1)
        sel_me = (lane1 == my_z)
        cnts = []
        for s in range(NZ):
            row = crecv_ref[s, :, :]
            cnts.append(jnp.sum(jnp.where(sel_me, row, 0)))
        offs = [jnp.int32(0)]
        for s in range(NZ - 1):
            offs.append(offs[-1] + cnts[s])

        jcol = lax.broadcasted_iota(jnp.int32, (T, 1), 0)
        qkey = jnp.zeros((T, 1), jnp.int32)
        for s in range(NZ):
            in_s = (jcol >= offs[s]) & (jcol < offs[s] + cnts[s])
            qkey = qkey + jnp.where(in_s, s * P_PAD + jcol - offs[s], 0)

        klane = lax.broadcasted_iota(jnp.int32, (T, P_PAD), 1)
        acc = jnp.zeros((T, D), jnp.float32)
        for off in range(NZ):
            s = lax.rem(my_z + off, NZ)
            if off > 0:
                pltpu.make_async_remote_copy(
                    src_ref=send_ref.at[pl.ds(s * P_PAD, P_PAD), :],
                    dst_ref=recv_ref.at[pl.ds(s * P_PAD, P_PAD), :],
                    send_sem=bsend_sems.at[s],
                    recv_sem=brecv_sems.at[s],
                    device_id=(my_x, my_y, s),
                    device_id_type=pl.DeviceIdType.MESH,
                ).wait_recv()
            q_s = (klane == qkey - s * P_PAD).astype(jnp.bfloat16)
            blk = recv_ref[pl.ds(s * P_PAD, P_PAD), :]
            acc = acc + jnp.dot(q_s, blk,
                                preferred_element_type=jnp.float32)
        out_ref[:, :] = acc

        for rdma in cnt_sends + blk_sends:
            rdma.wait_send()

    return pl.pallas_call(
        body,
        out_shape=jax.ShapeDtypeStruct((T, D), jnp.float32),
        in_specs=[
            pl.BlockSpec(memory_space=pltpu.VMEM),
            pl.BlockSpec(memory_space=pltpu.VMEM),
            pl.BlockSpec(memory_space=pltpu.VMEM),
        ],
        out_specs=pl.BlockSpec(memory_space=pltpu.VMEM),
        scratch_shapes=[
            pltpu.VMEM((NZ * P_PAD, D), jnp.bfloat16),
            pltpu.VMEM((NZ * P_PAD, D), jnp.bfloat16),
            pltpu.VMEM((1, 128), jnp.int32),
            pltpu.VMEM((NZ, 1, 128), jnp.int32),
            pltpu.SemaphoreType.DMA((NZ,)),
            pltpu.SemaphoreType.DMA((NZ,)),
            pltpu.SemaphoreType.DMA((NZ,)),
            pltpu.SemaphoreType.DMA((NZ,)),
        ],
        compiler_params=pltpu.CompilerParams(collective_id=0),
    )(x, d_row, d_col)
